# baseline (speedup 1.0000x reference)
import functools

import jax
import jax.numpy as jnp
from jax import lax
from jax.experimental import pallas as pl
from jax.experimental.pallas import tpu as pltpu

_MASK_THR = 0.45
_SIGMA = 2.0


def _sweep_body(T, nb, n, ma_ref, mb_ref, sa_ref, sb_ref, la_ref, lb_ref,
                m_ref, c_scr):
    b = pl.program_id(0)
    a = pl.program_id(1)

    @pl.when(a == 0)
    def _():
        m_ref[...] = jnp.zeros_like(m_ref)
        c_scr[:, pl.ds(b * T, T)] = jnp.zeros((1, T), jnp.float32)

    def tile(tri, edge):
        inter = lax.dot_general(ma_ref[...], mb_ref[...],
                                (((1,), (1,)), ((), ())),
                                preferred_element_type=jnp.float32)
        sa = sa_ref[0, 0, :]
        sb = sb_ref[0, 0, :]
        iou = inter / (sa[:, None] + sb[None, :] - inter)
        valid = la_ref[0, 0, :][:, None] == lb_ref[0, 0, :][None, :]
        if tri or edge:
            rj = lax.broadcasted_iota(jnp.int32, inter.shape, 1)
        if tri:
            ri = lax.broadcasted_iota(jnp.int32, inter.shape, 0)
            valid &= ri < rj
        if edge:
            valid &= (b * T + rj) < n
        d = jnp.where(valid, iou, 0.0)
        csl = c_scr[:, pl.ds(b * T, T)]
        csl = jnp.maximum(csl, jnp.max(d, axis=0)[None, :])
        c_scr[:, pl.ds(b * T, T)] = csl
        ca = c_scr[0, pl.ds(a * T, T)]
        term = d * d - (ca * ca)[:, None]
        m_ref[0, 0, :] = jnp.maximum(m_ref[0, 0, :], jnp.max(term, axis=0))

    edge_col = nb * T > n

    @pl.when(a < b)
    def _():
        if edge_col:
            @pl.when(b < nb - 1)
            def _():
                tile(False, False)

            @pl.when(b == nb - 1)
            def _():
                tile(False, True)
        else:
            tile(False, False)

    @pl.when(a == b)
    def _():
        if edge_col:
            @pl.when(b < nb - 1)
            def _():
                tile(True, False)

            @pl.when(b == nb - 1)
            def _():
                tile(True, True)
        else:
            tile(True, False)

    @pl.when(a == nb - 1)
    def _():
        m_ref[...] = jnp.exp(-_SIGMA * m_ref[...])


def _nms_core(masks, sums, labels, T):
    n, hw = masks.shape
    nb = (n + T - 1) // T
    P = nb * T
    grid = (nb, nb)
    sums3 = sums.reshape(nb, 1, T)
    labels3 = labels.reshape(nb, 1, T)
    mspec_a = pl.BlockSpec((T, hw), lambda b, a: (jnp.minimum(a, b), 0))
    mspec_b = pl.BlockSpec((T, hw), lambda b, a: (b, 0))
    vspec_a = pl.BlockSpec((1, 1, T), lambda b, a: (jnp.minimum(a, b), 0, 0))
    vspec_b = pl.BlockSpec((1, 1, T), lambda b, a: (b, 0, 0))
    params = pltpu.CompilerParams(dimension_semantics=("arbitrary", "arbitrary"))
    coeff = pl.pallas_call(
        functools.partial(_sweep_body, T, nb, n),
        grid=grid,
        in_specs=[mspec_a, mspec_b, vspec_a, vspec_b, vspec_a, vspec_b],
        out_specs=pl.BlockSpec((1, 1, T), lambda b, a: (b, 0, 0)),
        out_shape=jax.ShapeDtypeStruct((nb, 1, T), jnp.float32),
        scratch_shapes=[pltpu.VMEM((1, P), jnp.float32)],
        compiler_params=params,
    )(masks, masks, sums3, sums3, labels3, labels3)
    return coeff.reshape(P)


def kernel(seg_preds, cate_scores, cate_labels):
    n, h, w = seg_preds.shape
    hw = h * w
    seg_masks_b = seg_preds > _MASK_THR
    seg_masks_f = seg_masks_b.astype(jnp.float32)
    sum_masks = seg_masks_f.reshape(n, -1).sum(axis=1)
    seg_scores = (seg_preds * seg_masks_f).reshape(n, -1).sum(axis=1) / sum_masks
    cs = cate_scores * seg_scores
    sort_inds = jnp.argsort(-cs)

    T = 640
    P = ((n + T - 1) // T) * T
    pad = P - n

    masks_bf16 = seg_masks_f.astype(jnp.bfloat16)
    masks_s = jnp.take(masks_bf16, sort_inds, axis=0).reshape(n, hw)
    seg_preds_s = jnp.take(seg_preds, sort_inds, axis=0)
    sums_p = jnp.pad(jnp.take(sum_masks, sort_inds), (0, pad),
                     constant_values=1.0)
    labels_p = jnp.pad(jnp.take(cate_labels, sort_inds), (0, pad),
                       constant_values=-1)
    coeff = _nms_core(masks_s, sums_p, labels_p, T)
    scores_s = jnp.take(cs, sort_inds)
    return (seg_preds_s,
            scores_s * coeff[:n],
            jnp.take(cate_labels, sort_inds))
